# Initial kernel scaffold; baseline (speedup 1.0000x reference)
import jax
import jax.numpy as jnp
from jax.experimental import pallas as pl


def kernel(x0, edge_index, edge_vals, mask, scorer_0, Wz_0, Uz_0, bz_0, Wr_0, Ur_0, br_0, Wh_0, Uh_0, bh_0, W0_0, scorer_1, Wz_1, Uz_1, bz_1, Wr_1, Ur_1, br_1, Wh_1, Uh_1, bh_1, W0_1):
    raise NotImplementedError("write your pallas kernel here")


# speedup vs baseline: 3.5070x; 3.5070x over previous
import functools

import jax
import jax.numpy as jnp
from jax import lax
from jax.experimental import pallas as pl
from jax.experimental.pallas import tpu as pltpu
from jax.experimental.pallas import tpu_sc as plsc

T, N, E, D = 3, 10000, 320000, 128
NC, NS = 2, 16
NW = NC * NS
C = 128
NCH = 79
EPAD = NW * NCH * C
RSTRIDE = 624
RSPAN = 640

_NEG_INF = float('-inf')


def _bf(a):
    return a.astype(jnp.bfloat16)


def _dot1(a, b, dims):
    return lax.dot_general(_bf(a), _bf(b), (dims, ((), ())),
                           preferred_element_type=jnp.float32)


def _cell_and_project(ne_ref, mask_row_ref, scorer_ref, wz_ref, uz_ref, bz_ref,
                      wr_ref, ur_ref, br_ref, wh_ref, uh_ref, bh_ref,
                      wprev_ref, wnew_ref, x_ref, rows_ref):
    ne = ne_ref[...]
    scorer = scorer_ref[...]
    nrm = jnp.sqrt(jnp.sum(scorer * scorer))
    s = _dot1(scorer, ne, ((0,), (1,))) / nrm
    s = s + mask_row_ref[...]
    iota = lax.broadcasted_iota(jnp.int32, (1, N), 1)
    diota = lax.broadcasted_iota(jnp.int32, (D, 1), 0)

    def topk_step(k, carry):
        sc, vals = carry
        m = jnp.max(sc)
        idx = jnp.min(jnp.where(sc == m, iota, N))
        vals = jnp.where(diota == k, m, vals)
        rows_ref[pl.ds(k, 1), :] = ne_ref[pl.ds(idx, 1), :]
        sc = jnp.where(iota == idx, _NEG_INF, sc)
        return sc, vals

    vals0 = jnp.zeros((D, 1), jnp.float32)
    _, vals = lax.fori_loop(0, D, topk_step, (s, vals0))
    zt = rows_ref[...] * jnp.tanh(vals)

    def gdot(wm_ref):
        return _dot1(wm_ref[...], zt, ((1,), (1,)))

    wp = wprev_ref[...]
    upd = jax.nn.sigmoid(gdot(wz_ref) + _dot1(uz_ref[...], wp, ((1,), (0,))) + bz_ref[...])
    rst = jax.nn.sigmoid(gdot(wr_ref) + _dot1(ur_ref[...], wp, ((1,), (0,))) + br_ref[...])
    hcap = jnp.tanh(gdot(wh_ref) + _dot1(uh_ref[...], rst * wp, ((1,), (0,))) + bh_ref[...])
    wn = (1.0 - upd) * wp + upd * hcap
    wnew_ref[...] = wn
    x_ref[...] = _dot1(ne, wn, ((1,), (0,)))


def _dense_x0_body(ne_ref, *rest):
    _cell_and_project(ne_ref, *rest[:-1])


def _dense_p_body(p0_ref, p1_ref, *rest):
    ne_s = rest[-1]
    ne_s[...] = jnp.maximum(p0_ref[...] + p1_ref[...], 0.0)
    _cell_and_project(ne_s, *rest[:-1])


def _dense_call(body, ne_args, mask_row, p, wprev):
    out = pl.pallas_call(
        body,
        out_shape=[jax.ShapeDtypeStruct((D, D), jnp.float32),
                   jax.ShapeDtypeStruct((N, D), jnp.float32)],
        scratch_shapes=[pltpu.VMEM((D, D), jnp.float32),
                        pltpu.VMEM((N, D), jnp.float32)],
    )(*ne_args, mask_row, p['scorer'], p['Wz'], p['Uz'], p['bz'],
      p['Wr'], p['Ur'], p['br'], p['Wh'], p['Uh'], p['bh'], wprev)
    return out[0], out[1]


def _spmm_body(x_hbm, src_hbm, dst_hbm, ev_hbm, out_hbm,
               src_v, dst_v, ev_v, rows_v, y_sp, sem):
    c = lax.axis_index("c")
    s = lax.axis_index("s")
    w = s * NC + c

    z16 = jnp.zeros((16,), jnp.float32)

    def zrow(r, _):
        for q in range(8):
            rows_v[r, pl.ds(16 * q, 16)] = z16
        return 0

    lax.fori_loop(0, C, zrow, 0)
    base = s * RSTRIDE
    for i in range(RSPAN // C):
        pltpu.sync_copy(rows_v, y_sp.at[pl.ds(base + i * C, C)])
    plsc.subcore_barrier()

    pltpu.sync_copy(src_hbm.at[w], src_v)
    pltpu.sync_copy(dst_hbm.at[w], dst_v)
    pltpu.sync_copy(ev_hbm.at[w], ev_v)

    def chunk(j, _):
        pltpu.async_copy(x_hbm.at[src_v.at[j]], rows_v, sem).wait()

        def groupscale(g, _):
            evv = ev_v[j, pl.ds(g * 16, 16)]
            for r16 in range(16):
                ev_s = evv[r16]
                r = g * 16 + r16
                for q in range(8):
                    sl = pl.ds(16 * q, 16)
                    rows_v[r, sl] = rows_v[r, sl] * ev_s
            return 0

        lax.fori_loop(0, C // 16, groupscale, 0)
        pltpu.sync_copy(rows_v, y_sp.at[dst_v.at[j]], add=True)
        return 0

    lax.fori_loop(0, NCH, chunk, 0)
    plsc.subcore_barrier()
    pltpu.sync_copy(y_sp.at[pl.ds(base, RSPAN)],
                    out_hbm.at[c, pl.ds(base, RSPAN)])


def _spmm(x, src, dst, ev):
    mesh = plsc.VectorSubcoreMesh(core_axis_name="c", subcore_axis_name="s")
    f = pl.kernel(
        _spmm_body,
        out_type=jax.ShapeDtypeStruct((NC, N, D), jnp.float32),
        mesh=mesh,
        scratch_types=[
            pltpu.VMEM((NCH, C), jnp.int32),
            pltpu.VMEM((NCH, C), jnp.int32),
            pltpu.VMEM((NCH, C), jnp.float32),
            pltpu.VMEM((C, D), jnp.float32),
            pltpu.VMEM_SHARED((N, D), jnp.float32),
            pltpu.SemaphoreType.DMA,
        ],
    )
    return f(x, src, dst, ev)


def _relu_add_body(p0_ref, p1_ref, o_ref):
    o_ref[...] = jnp.maximum(p0_ref[...] + p1_ref[...], 0.0)


def _relu_add(parts):
    return pl.pallas_call(
        _relu_add_body,
        out_shape=jax.ShapeDtypeStruct((N, D), jnp.float32),
    )(parts[0], parts[1])


def kernel(x0, edge_index, edge_vals, mask, scorer_0, Wz_0, Uz_0, bz_0,
           Wr_0, Ur_0, br_0, Wh_0, Uh_0, bh_0, W0_0, scorer_1, Wz_1, Uz_1,
           bz_1, Wr_1, Ur_1, br_1, Wh_1, Uh_1, bh_1, W0_1):
    ei = jnp.pad(edge_index, ((0, 0), (0, 0), (0, EPAD - E)))
    ev = jnp.pad(edge_vals, ((0, 0), (0, EPAD - E)))
    src = ei[:, 1].reshape(T, NW, NCH, C)
    dst = ei[:, 0].reshape(T, NW, NCH, C)
    evr = ev.reshape(T, NW, NCH, C)
    mask_row = mask.reshape(T, 1, N)

    l0 = {'scorer': scorer_0, 'Wz': Wz_0, 'Uz': Uz_0, 'bz': bz_0,
          'Wr': Wr_0, 'Ur': Ur_0, 'br': br_0, 'Wh': Wh_0, 'Uh': Uh_0,
          'bh': bh_0, 'W0': W0_0}
    l1 = {'scorer': scorer_1, 'Wz': Wz_1, 'Uz': Uz_1, 'bz': bz_1,
          'Wr': Wr_1, 'Ur': Ur_1, 'br': br_1, 'Wh': Wh_1, 'Uh': Uh_1,
          'bh': bh_1, 'W0': W0_1}

    parts = [None] * T
    for l, p in ((0, l0), (1, l1)):
        w = p['W0']
        newparts = [None] * T
        for t in range(T):
            if l == 0:
                w, x = _dense_call(_dense_x0_body, (x0[t],), mask_row[t], p, w)
            else:
                w, x = _dense_call(_dense_p_body, (parts[t][0], parts[t][1]),
                                   mask_row[t], p, w)
            newparts[t] = _spmm(x, src[t], dst[t], evr[t])
        parts = newparts

    return _relu_add(parts[T - 1])
